# baseline (speedup 1.0000x reference)
import functools

import jax
import jax.numpy as jnp
from jax import lax
from jax.experimental import pallas as pl
from jax.experimental.pallas import tpu as pltpu
from jax.experimental.pallas import tpu_sc as plsc

N = 10000
E = 320000
EMB = 128
OUT = 256
NDENSE = 3
NT = 12
RBF = 6


_BE = 16000


def _edge_body(rbfT_ref, x_ref, wrbf_ref, xg_ref):
    g = lax.dot_general(
        rbfT_ref[...],
        wrbf_ref[...],
        dimension_numbers=(((0,), (0,)), ((), ())),
        preferred_element_type=jnp.float32,
    )
    xg_ref[...] = g * x_ref[...]


def _edge_stage(rbf_T, x, W_rbf):
    return pl.pallas_call(
        _edge_body,
        grid=(E // _BE,),
        in_specs=[
            pl.BlockSpec((RBF, _BE), lambda i: (0, i)),
            pl.BlockSpec((_BE, EMB), lambda i: (i, 0)),
            pl.BlockSpec((RBF, EMB), lambda i: (0, 0)),
        ],
        out_specs=pl.BlockSpec((_BE, EMB), lambda i: (i, 0)),
        out_shape=jax.ShapeDtypeStruct((E, EMB), jnp.float32),
    )(rbf_T, x, W_rbf)


_NW = 32
_IR = E // 128
_IRT = 80
_IRT_LAST = _IR - 31 * _IRT
_CH = 128
_NCH = _IRT
_NCH_LAST = _IRT_LAST
_RPT = 624
_TAIL0 = 16 * _RPT


def _scatter_body(xg_hbm, idx_hbm, out_hbm, idx_v, buf0, buf1, sem0, sem1,
                  pooled_sh):
    c = lax.axis_index("c")
    s = lax.axis_index("s")
    w = s * 2 + c

    def _zrow(i, carry):
        for cc in range(EMB // 16):
            buf0[i, pl.ds(cc * 16, 16)] = jnp.zeros((16,), jnp.float32)
        return carry

    lax.fori_loop(0, _CH, _zrow, 0)

    for k in range(4):
        pltpu.sync_copy(buf0, pooled_sh.at[pl.ds(s * _RPT + k * _CH, _CH)])
    pltpu.sync_copy(
        buf0.at[pl.ds(0, _RPT - 4 * _CH)],
        pooled_sh.at[pl.ds(s * _RPT + 4 * _CH, _RPT - 4 * _CH)],
    )

    @pl.when(s == 15)
    def _():
        pltpu.sync_copy(buf0.at[pl.ds(0, 16)], pooled_sh.at[pl.ds(_TAIL0, 16)])

    plsc.subcore_barrier()

    @pl.when(w < 31)
    def _():
        pltpu.sync_copy(idx_hbm.at[pl.ds(w * _IRT, _IRT)], idx_v)

    @pl.when(w == 31)
    def _():
        pltpu.sync_copy(
            idx_hbm.at[pl.ds(31 * _IRT, _IRT_LAST)], idx_v.at[pl.ds(0, _IRT_LAST)]
        )

    nch = jnp.where(w == 31, _NCH_LAST, _NCH)
    edge_base = w * (_IRT * 128)

    def _load(i, buf, sem):
        pltpu.make_async_copy(
            xg_hbm.at[pl.ds(edge_base + i * _CH, _CH)], buf, sem
        ).start()

    def _wait(i, buf, sem):
        pltpu.make_async_copy(
            xg_hbm.at[pl.ds(edge_base + i * _CH, _CH)], buf, sem
        ).wait()

    def _scat(i, buf):
        pltpu.sync_copy(buf, pooled_sh.at[idx_v.at[i]], add=True)

    _load(0, buf0, sem0)

    def _pair(t, carry):
        i0 = 2 * t
        i1 = i0 + 1

        @pl.when(i1 < nch)
        def _():
            _load(i1, buf1, sem1)

        _wait(i0, buf0, sem0)
        _scat(i0, buf0)

        @pl.when(i1 + 1 < nch)
        def _():
            _load(i1 + 1, buf0, sem0)

        @pl.when(i1 < nch)
        def _():
            _wait(i1, buf1, sem1)
            _scat(i1, buf1)

        return carry

    lax.fori_loop(0, (nch + 1) // 2, _pair, 0)
    plsc.subcore_barrier()

    pltpu.sync_copy(
        pooled_sh.at[pl.ds(s * _RPT, _RPT)], out_hbm.at[c, pl.ds(s * _RPT, _RPT)]
    )

    @pl.when(s == 15)
    def _():
        pltpu.sync_copy(
            pooled_sh.at[pl.ds(_TAIL0, 16)], out_hbm.at[c, pl.ds(_TAIL0, 16)]
        )


def _scatter_stage(xg, idx2d):
    mesh = plsc.VectorSubcoreMesh(core_axis_name="c", subcore_axis_name="s")
    f = functools.partial(
        pl.kernel,
        mesh=mesh,
        out_type=jax.ShapeDtypeStruct((2, N, EMB), jnp.float32),
        scratch_types=[
            pltpu.VMEM((_IRT, 128), jnp.int32),
            pltpu.VMEM((_CH, EMB), jnp.float32),
            pltpu.VMEM((_CH, EMB), jnp.float32),
            pltpu.SemaphoreType.DMA,
            pltpu.SemaphoreType.DMA,
            pltpu.VMEM_SHARED((N, EMB), jnp.float32),
        ],
    )(_scatter_body)
    return f(xg, idx2d)


_BN = 1000


def _node_body(p0_ref, p1_ref, wup_ref, wmlp_ref, bmlp_ref, wout_ref, out_ref):
    pooled = p0_ref[0] + p1_ref[0]
    h = jnp.dot(pooled, wup_ref[...], preferred_element_type=jnp.float32)
    for i in range(NDENSE):
        z = jnp.dot(h, wmlp_ref[i], preferred_element_type=jnp.float32)
        z = z + bmlp_ref[i][None, :]
        h = z * (1.0 / (1.0 + jnp.exp(-z)))
    out_ref[...] = jnp.dot(h, wout_ref[...], preferred_element_type=jnp.float32)


def _node_stage(partial, W_up, W_mlp, b_mlp, W_out):
    return pl.pallas_call(
        _node_body,
        grid=(N // _BN,),
        in_specs=[
            pl.BlockSpec((1, _BN, EMB), lambda i: (0, i, 0)),
            pl.BlockSpec((1, _BN, EMB), lambda i: (1, i, 0)),
            pl.BlockSpec((EMB, OUT), lambda i: (0, 0)),
            pl.BlockSpec((NDENSE, OUT, OUT), lambda i: (0, 0, 0)),
            pl.BlockSpec((NDENSE, OUT), lambda i: (0, 0)),
            pl.BlockSpec((OUT, NT), lambda i: (0, 0)),
        ],
        out_specs=pl.BlockSpec((_BN, NT), lambda i: (i, 0)),
        out_shape=jax.ShapeDtypeStruct((N, NT), jnp.float32),
    )(partial, partial, W_up, W_mlp, b_mlp, W_out)


def kernel(n_atoms, x, rbf, tensor_index, W_rbf, W_up, W_mlp, b_mlp, W_out):
    del n_atoms
    idx2d = tensor_index.astype(jnp.int32).reshape(E // 128, 128)
    xg = _edge_stage(rbf.T, x, W_rbf)
    partial = _scatter_stage(xg, idx2d)
    return _node_stage(partial, W_up, W_mlp, b_mlp, W_out)


# speedup vs baseline: 1.0005x; 1.0005x over previous
import functools

import jax
import jax.numpy as jnp
from jax import lax
from jax.experimental import pallas as pl
from jax.experimental.pallas import tpu as pltpu
from jax.experimental.pallas import tpu_sc as plsc

N = 10000
E = 320000
EMB = 128
OUT = 256
NDENSE = 3
NT = 12
RBF = 6


_BE = 12800


def _edge_body(rbfT_ref, x_ref, wrbf_ref, xg_ref):
    g = lax.dot_general(
        rbfT_ref[...],
        wrbf_ref[...],
        dimension_numbers=(((0,), (0,)), ((), ())),
        preferred_element_type=jnp.float32,
    )
    xg_ref[...] = g * x_ref[...]


def _edge_stage(rbf_T, x, W_rbf):
    return pl.pallas_call(
        _edge_body,
        grid=(E // _BE,),
        in_specs=[
            pl.BlockSpec((RBF, _BE), lambda i: (0, i)),
            pl.BlockSpec((_BE, EMB), lambda i: (i, 0)),
            pl.BlockSpec((RBF, EMB), lambda i: (0, 0)),
        ],
        out_specs=pl.BlockSpec((_BE, EMB), lambda i: (i, 0)),
        out_shape=jax.ShapeDtypeStruct((E, EMB), jnp.float32),
    )(rbf_T, x, W_rbf)


_NW = 32
_IR = E // 128
_IRT = 80
_IRT_LAST = _IR - 31 * _IRT
_CH = 128
_NCH = _IRT
_NCH_LAST = _IRT_LAST
_RPT = 624
_TAIL0 = 16 * _RPT


def _scatter_body(xg_hbm, idx_hbm, out_hbm, idx_v, buf0, buf1, sem0, sem1,
                  pooled_sh):
    c = lax.axis_index("c")
    s = lax.axis_index("s")
    w = s * 2 + c

    def _zrow(i, carry):
        for cc in range(EMB // 16):
            buf0[i, pl.ds(cc * 16, 16)] = jnp.zeros((16,), jnp.float32)
        return carry

    lax.fori_loop(0, _CH, _zrow, 0)

    for k in range(4):
        pltpu.sync_copy(buf0, pooled_sh.at[pl.ds(s * _RPT + k * _CH, _CH)])
    pltpu.sync_copy(
        buf0.at[pl.ds(0, _RPT - 4 * _CH)],
        pooled_sh.at[pl.ds(s * _RPT + 4 * _CH, _RPT - 4 * _CH)],
    )

    @pl.when(s == 15)
    def _():
        pltpu.sync_copy(buf0.at[pl.ds(0, 16)], pooled_sh.at[pl.ds(_TAIL0, 16)])

    plsc.subcore_barrier()

    @pl.when(w < 31)
    def _():
        pltpu.sync_copy(idx_hbm.at[pl.ds(w * _IRT, _IRT)], idx_v)

    @pl.when(w == 31)
    def _():
        pltpu.sync_copy(
            idx_hbm.at[pl.ds(31 * _IRT, _IRT_LAST)], idx_v.at[pl.ds(0, _IRT_LAST)]
        )

    nch = jnp.where(w == 31, _NCH_LAST, _NCH)
    edge_base = w * (_IRT * 128)

    def _load(i, buf, sem):
        pltpu.make_async_copy(
            xg_hbm.at[pl.ds(edge_base + i * _CH, _CH)], buf, sem
        ).start()

    def _wait(i, buf, sem):
        pltpu.make_async_copy(
            xg_hbm.at[pl.ds(edge_base + i * _CH, _CH)], buf, sem
        ).wait()

    def _scat(i, buf):
        pltpu.sync_copy(buf, pooled_sh.at[idx_v.at[i]], add=True)

    _load(0, buf0, sem0)

    def _pair(t, carry):
        i0 = 2 * t
        i1 = i0 + 1

        @pl.when(i1 < nch)
        def _():
            _load(i1, buf1, sem1)

        _wait(i0, buf0, sem0)
        _scat(i0, buf0)

        @pl.when(i1 + 1 < nch)
        def _():
            _load(i1 + 1, buf0, sem0)

        @pl.when(i1 < nch)
        def _():
            _wait(i1, buf1, sem1)
            _scat(i1, buf1)

        return carry

    lax.fori_loop(0, (nch + 1) // 2, _pair, 0)
    plsc.subcore_barrier()

    pltpu.sync_copy(
        pooled_sh.at[pl.ds(s * _RPT, _RPT)], out_hbm.at[c, pl.ds(s * _RPT, _RPT)]
    )

    @pl.when(s == 15)
    def _():
        pltpu.sync_copy(
            pooled_sh.at[pl.ds(_TAIL0, 16)], out_hbm.at[c, pl.ds(_TAIL0, 16)]
        )


def _scatter_stage(xg, idx2d):
    mesh = plsc.VectorSubcoreMesh(core_axis_name="c", subcore_axis_name="s")
    f = functools.partial(
        pl.kernel,
        mesh=mesh,
        out_type=jax.ShapeDtypeStruct((2, N, EMB), jnp.float32),
        scratch_types=[
            pltpu.VMEM((_IRT, 128), jnp.int32),
            pltpu.VMEM((_CH, EMB), jnp.float32),
            pltpu.VMEM((_CH, EMB), jnp.float32),
            pltpu.SemaphoreType.DMA,
            pltpu.SemaphoreType.DMA,
            pltpu.VMEM_SHARED((N, EMB), jnp.float32),
        ],
    )(_scatter_body)
    return f(xg, idx2d)


_BN = 1000


def _node_body(p0_ref, p1_ref, wup_ref, wmlp_ref, bmlp_ref, wout_ref, out_ref):
    pooled = p0_ref[0] + p1_ref[0]
    h = jnp.dot(pooled, wup_ref[...], preferred_element_type=jnp.float32)
    for i in range(NDENSE):
        z = jnp.dot(h, wmlp_ref[i], preferred_element_type=jnp.float32)
        z = z + bmlp_ref[i][None, :]
        h = z * (1.0 / (1.0 + jnp.exp(-z)))
    out_ref[...] = jnp.dot(h, wout_ref[...], preferred_element_type=jnp.float32)


def _node_stage(partial, W_up, W_mlp, b_mlp, W_out):
    return pl.pallas_call(
        _node_body,
        grid=(N // _BN,),
        in_specs=[
            pl.BlockSpec((1, _BN, EMB), lambda i: (0, i, 0)),
            pl.BlockSpec((1, _BN, EMB), lambda i: (1, i, 0)),
            pl.BlockSpec((EMB, OUT), lambda i: (0, 0)),
            pl.BlockSpec((NDENSE, OUT, OUT), lambda i: (0, 0, 0)),
            pl.BlockSpec((NDENSE, OUT), lambda i: (0, 0)),
            pl.BlockSpec((OUT, NT), lambda i: (0, 0)),
        ],
        out_specs=pl.BlockSpec((_BN, NT), lambda i: (i, 0)),
        out_shape=jax.ShapeDtypeStruct((N, NT), jnp.float32),
    )(partial, partial, W_up, W_mlp, b_mlp, W_out)


def kernel(n_atoms, x, rbf, tensor_index, W_rbf, W_up, W_mlp, b_mlp, W_out):
    del n_atoms
    idx2d = tensor_index.astype(jnp.int32).reshape(E // 128, 128)
    xg = _edge_stage(rbf.T, x, W_rbf)
    partial = _scatter_stage(xg, idx2d)
    return _node_stage(partial, W_up, W_mlp, b_mlp, W_out)
